# baseline (speedup 1.0000x reference)
import functools

import jax
import jax.numpy as jnp
from jax import lax
from jax.experimental import pallas as pl
from jax.experimental.pallas import tpu as pltpu
from jax.experimental.pallas import tpu_sc as plsc

N_LEVELS = 8
PS = 512
FD = 16
NPTS = 262144
MB = (0, 262144, 327680, 344064, 348160, 349184, 349440, 349504)
PLANE_TEXELS = 349520
T_ROWS = 3 * PLANE_TEXELS
NC = 2
NS = 16
F32 = jnp.float32
I32 = jnp.int32


def _mesh():
    return plsc.VectorSubcoreMesh(core_axis_name="c", subcore_axis_name="s")


_PARAMS = pltpu.CompilerParams(use_tc_tiling_on_sc=False,
                               needs_layout_passes=False)


def _build_body(tex_ref, table_ref, l0buf, buf0, buf1, obuf, sem):
    cid = lax.axis_index("c")
    sid = lax.axis_index("s")


    for p in range(3):
        def l0_batch(i, _, p=p):
            y0 = sid * 32 + i * 8
            h0 = pltpu.async_copy(
                tex_ref.at[p, pl.ds(y0, 8), pl.ds(cid * 256, 256)],
                l0buf.at[:, pl.ds(0, 256)], sem)
            ecol = jnp.minimum(cid * 256 + 256, 511)
            h1 = pltpu.async_copy(
                tex_ref.at[p, pl.ds(y0, 8), pl.ds(ecol, 1)],
                l0buf.at[:, pl.ds(256, 1)], sem)
            h0.wait()
            h1.wait()
            writes = []
            for r in range(8):
                dst = p * PLANE_TEXELS + (y0 + r) * 512 + cid * 256
                writes.append(pltpu.async_copy(
                    l0buf.at[r, pl.ds(0, 256)],
                    table_ref.at[pl.ds(dst, 256), pl.ds(0, FD)], sem))
                writes.append(pltpu.async_copy(
                    l0buf.at[r, pl.ds(1, 256)],
                    table_ref.at[pl.ds(dst, 256), pl.ds(FD, FD)], sem))
            for wco in writes:
                wco.wait()
            return 0
        lax.fori_loop(0, 4, l0_batch, 0)
    plsc.subcore_barrier()

    for m in range(1, N_LEVELS):
        S = PS >> m
        W = S // 2
        Sin = S * 2
        rpt = max(S // NS, 1)
        for p in range(3):
            pbin = p * PLANE_TEXELS + MB[m - 1]
            pbout = p * PLANE_TEXELS + MB[m]

            def row_body(y, _, S=S, W=W, Sin=Sin, pbin=pbin, pbout=pbout):
                cin = cid * S
                src0 = pbin + (2 * y) * Sin + cin
                c0 = pltpu.async_copy(
                    table_ref.at[pl.ds(src0, S + 2), pl.ds(0, FD)],
                    buf0.at[pl.ds(0, S + 2)], sem)
                c1 = pltpu.async_copy(
                    table_ref.at[pl.ds(src0 + Sin, S + 2), pl.ds(0, FD)],
                    buf1.at[pl.ds(0, S + 2)], sem)
                c0.wait()
                c1.wait()
                for c in range(W + 1):
                    v = (buf0[2 * c, :] + buf1[2 * c, :]
                         + buf0[2 * c + 1, :] + buf1[2 * c + 1, :]) * 0.25
                    obuf[c, :] = v

                @pl.when(cid == 1)
                def _():
                    obuf[W, :] = obuf[W - 1, :]

                dst = pbout + y * S + cid * W
                w0 = pltpu.async_copy(
                    obuf.at[pl.ds(0, W)],
                    table_ref.at[pl.ds(dst, W), pl.ds(0, FD)], sem)
                w1 = pltpu.async_copy(
                    obuf.at[pl.ds(1, W)],
                    table_ref.at[pl.ds(dst, W), pl.ds(FD, FD)], sem)
                w0.wait()
                w1.wait()
                return 0

            lo = jnp.minimum(sid * rpt, S)
            hi = jnp.minimum((sid + 1) * rpt, S)
            lax.fori_loop(lo, hi, row_body, 0)
        plsc.subcore_barrier()


def _build(tex4):
    kern = pl.kernel(
        _build_body,
        out_type=jax.ShapeDtypeStruct((T_ROWS, 2 * FD), F32),
        mesh=_mesh(),
        compiler_params=_PARAMS,
        scratch_types=[
            pltpu.VMEM((8, 257, FD), F32),
            pltpu.VMEM((258, FD), F32),
            pltpu.VMEM((258, FD), F32),
            pltpu.VMEM((130, FD), F32),
            pltpu.SemaphoreType.DMA,
        ],
    )
    return kern(tex4)


BLK = 128
NBLK = NPTS // (NC * NS * BLK)


def _splat(val, dtype=I32):
    return jnp.full((16,), val, dtype)


def _sample_body(pts_ref, table_ref, out_ref,
                 pts_v, idx_v, w_v, rows_v, out_v, sem, gsem):
    cid = lax.axis_index("c")
    sid = lax.axis_index("s")
    wid = sid * NC + cid
    base0 = wid * (NBLK * BLK)

    def phase_a(blk, par):
        b0 = base0 + blk * BLK
        ld = [pltpu.async_copy(pts_ref.at[c, pl.ds(b0, BLK)], pts_v.at[c],
                               sem) for c in range(4)]
        for h in ld:
            h.wait()

        def grp_a(g, _):
            s = g * 16
            xv = pts_v[0, pl.ds(s, 16)]
            yv = pts_v[1, pl.ds(s, 16)]
            zv = pts_v[2, pl.ds(s, 16)]
            lv = pts_v[3, pl.ds(s, 16)]
            lvc = jnp.minimum(jnp.maximum(lv, 0.0), 7.0)
            m0 = lvc.astype(I32)
            fl = lvc - m0.astype(F32)
            m1 = jnp.minimum(m0 + 1, 7)
            for mipslot, (mm, wm) in enumerate(((m0, 1.0 - fl), (m1, fl))):
                S = jnp.right_shift(_splat(PS), mm)
                Sf = S.astype(F32)
                Sm1 = S - 1
                mbv = _splat(0)
                for m in range(1, N_LEVELS):
                    mbv = jnp.where(mm == m, MB[m], mbv)
                cc = []
                for u in (xv, yv, zv):
                    pos = u * Sf - 0.5
                    ti = pos.astype(I32)
                    tf = ti.astype(F32)
                    neg = tf > pos
                    f0 = jnp.where(neg, ti - 1, ti)
                    f0f = jnp.where(neg, tf - 1.0, tf)
                    frac = pos - f0f
                    e0 = jnp.minimum(jnp.maximum(f0, 0), Sm1)
                    e1 = jnp.minimum(e0 + 1, Sm1)
                    cc.append((e0, e1, frac))
                for p, (ui, vi) in enumerate(((1, 2), (0, 2), (0, 1))):
                    ux0, ux1, fu = cc[ui]
                    vy0, vy1, fv = cc[vi]
                    pb = mbv + p * PLANE_TEXELS
                    r0 = pb + vy0 * S
                    r1 = pb + vy1 * S
                    wa = (1.0 - fv) * wm
                    wb = fv * wm
                    gu = 1.0 - fu
                    k = p * 8 + mipslot * 4
                    q = p * 4 + mipslot * 2
                    idx_v[par, q + 0, pl.ds(s, 16)] = r0 + ux0
                    idx_v[par, q + 1, pl.ds(s, 16)] = r1 + ux0
                    w_v[par, k + 0, pl.ds(s, 16)] = gu * wa
                    w_v[par, k + 1, pl.ds(s, 16)] = fu * wa
                    w_v[par, k + 2, pl.ds(s, 16)] = gu * wb
                    w_v[par, k + 3, pl.ds(s, 16)] = fu * wb
            return 0

        lax.fori_loop(0, 8, grp_a, 0)

    def fire(par):
        for k in range(12):
            pltpu.async_copy(table_ref.at[idx_v.at[par, k]],
                             rows_v.at[par, k], gsem)

    def drain(par):
        for k in range(12):
            pltpu.make_async_copy(table_ref.at[idx_v.at[par, k]],
                                  rows_v.at[par, k], gsem).wait()

    def phase_b(blk, par):
        def grp_b(g, _):
            s = g * 16
            for p in range(3):
                wk = [w_v[par, p * 8 + j, pl.ds(s, 16)] for j in range(8)]
                for i in range(16):
                    acc = None
                    for qq in range(4):
                        lo = rows_v[par, p * 4 + qq, s + i, pl.ds(0, FD)]
                        hi = rows_v[par, p * 4 + qq, s + i, pl.ds(FD, FD)]
                        term = wk[2 * qq][i] * lo + wk[2 * qq + 1][i] * hi
                        acc = term if acc is None else acc + term
                    out_v[s + i, pl.ds(p * 16, 16)] = acc
            return 0

        lax.fori_loop(0, 8, grp_b, 0)
        b0 = base0 + blk * BLK
        pltpu.async_copy(out_v, out_ref.at[pl.ds(b0, BLK)], sem).wait()

    phase_a(0, 0)
    fire(0)

    def pair(i, _):
        blk0 = 2 * i
        phase_a(blk0 + 1, 1)
        fire(1)
        drain(0)
        phase_b(blk0, 0)

        @pl.when(i < NBLK // 2 - 1)
        def _():
            phase_a(blk0 + 2, 0)
            fire(0)

        drain(1)
        phase_b(blk0 + 1, 1)
        return 0

    lax.fori_loop(0, NBLK // 2, pair, 0)


def _sample(pts, table):
    kern = pl.kernel(
        _sample_body,
        out_type=jax.ShapeDtypeStruct((NPTS, 48), F32),
        mesh=_mesh(),
        compiler_params=_PARAMS,
        scratch_types=[
            pltpu.VMEM((4, BLK), F32),
            pltpu.VMEM((2, 12, BLK), I32),
            pltpu.VMEM((2, 24, BLK), F32),
            pltpu.VMEM((2, 12, BLK, 2 * FD), F32),
            pltpu.VMEM((BLK, 48), F32),
            pltpu.SemaphoreType.DMA,
            pltpu.SemaphoreType.DMA,
        ],
    )
    return kern(pts, table)


@jax.jit
def kernel(x, level, texture):
    pts = jnp.maximum(
        jnp.concatenate([x.T, level[None, :]], axis=0).astype(F32), -1e30)
    tex = jnp.maximum(texture, -1e30)
    table = _build(tex)
    return _sample(pts, table)


# speedup vs baseline: 1.2351x; 1.2351x over previous
import functools

import jax
import jax.numpy as jnp
from jax import lax
from jax.experimental import pallas as pl
from jax.experimental.pallas import tpu as pltpu
from jax.experimental.pallas import tpu_sc as plsc

N_LEVELS = 8
PS = 512
FD = 16
NPTS = 262144
MB = (0, 262144, 327680, 344064, 348160, 349184, 349440, 349504)
PLANE_TEXELS = 349520
T_ROWS = 3 * PLANE_TEXELS
NC = 2
NS = 16
F32 = jnp.float32
I32 = jnp.int32


def _mesh():
    return plsc.VectorSubcoreMesh(core_axis_name="c", subcore_axis_name="s")


_PARAMS = pltpu.CompilerParams(use_tc_tiling_on_sc=False,
                               needs_layout_passes=False)


def _build_body(tex_ref, table_ref, tbuf, rbuf, obuf, rsem, wsem):
    cid = lax.axis_index("c")
    sid = lax.axis_index("s")


    def l0_read(t, par):
        p = t // 4
        y0 = sid * 32 + (t - p * 4) * 8
        pltpu.async_copy(
            tex_ref.at[p, pl.ds(y0, 8), pl.ds(cid * 256, 256)],
            tbuf.at[par, :, pl.ds(0, 256)], rsem)
        ecol = jnp.minimum(cid * 256 + 256, 511)
        pltpu.async_copy(
            tex_ref.at[p, pl.ds(y0, 8), pl.ds(ecol, 1)],
            tbuf.at[par, :, pl.ds(256, 1)], rsem)

    def l0_rwait(t, par):
        p = t // 4
        y0 = sid * 32 + (t - p * 4) * 8
        pltpu.make_async_copy(
            tex_ref.at[p, pl.ds(y0, 8), pl.ds(cid * 256, 256)],
            tbuf.at[par, :, pl.ds(0, 256)], rsem).wait()
        pltpu.make_async_copy(
            tex_ref.at[p, pl.ds(y0, 8), pl.ds(0, 1)],
            tbuf.at[par, :, pl.ds(256, 1)], rsem).wait()

    def l0_wfire(t, par, drain):
        p = t // 4
        y0 = sid * 32 + (t - p * 4) * 8
        for r in range(8):
            dst = p * PLANE_TEXELS + (y0 + r) * 512 + cid * 256
            for sl, c0 in ((0, 0), (FD, 1)):
                cp = pltpu.make_async_copy(
                    tbuf.at[par, r, pl.ds(c0, 256)],
                    table_ref.at[pl.ds(dst, 256), pl.ds(sl, FD)], wsem)
                if drain:
                    cp.wait()
                else:
                    cp.start()

    l0_read(0, 0)

    def l0_body(t, _):
        par = t & 1

        @pl.when(t > 0)
        def _():
            l0_wfire(t - 1, 1 - par, True)

        @pl.when(t < 11)
        def _():
            l0_read(t + 1, 1 - par)

        l0_rwait(t, par)
        l0_wfire(t, par, False)
        return 0

    lax.fori_loop(0, 12, l0_body, 0)
    l0_wfire(11, 1, True)
    plsc.subcore_barrier()

    for m in range(1, N_LEVELS):
        S = PS >> m
        W = S // 2
        Sin = S * 2
        rpt = max(S // NS, 1)
        lo = jnp.minimum(sid * rpt, S)
        hi = jnp.minimum((sid + 1) * rpt, S)
        n = hi - lo
        total = 3 * n

        def py_of(t, n=n, lo=lo):
            p = t // jnp.maximum(n, 1)
            return p, lo + t - p * n

        def rd(t, par, go, S=S, Sin=Sin, m=m):
            p, y = py_of(t)
            pbin = p * PLANE_TEXELS + MB[m - 1]
            src0 = pbin + (2 * y) * Sin + cid * S
            for half in range(2):
                cp = pltpu.make_async_copy(
                    table_ref.at[pl.ds(src0 + half * Sin, S + 2),
                                 pl.ds(0, FD)],
                    rbuf.at[par, half, pl.ds(0, S + 2)], rsem)
                if go:
                    cp.start()
                else:
                    cp.wait()

        def wr(t, par, drain, S=S, W=W, m=m):
            p, y = py_of(t)
            dst = p * PLANE_TEXELS + MB[m] + y * S + cid * W
            for sl, c0 in ((0, 0), (FD, 1)):
                cp = pltpu.make_async_copy(
                    obuf.at[par, pl.ds(c0, W)],
                    table_ref.at[pl.ds(dst, W), pl.ds(sl, FD)], wsem)
                if drain:
                    cp.wait()
                else:
                    cp.start()

        def lvl_body(t, _, S=S, W=W):
            par = 0

            rd(t, par, True)
            rd(t, par, False)
            for c in range(W + 1):
                v = (rbuf[par, 0, 2 * c, :] + rbuf[par, 1, 2 * c, :]
                     + rbuf[par, 0, 2 * c + 1, :]
                     + rbuf[par, 1, 2 * c + 1, :]) * 0.25
                obuf[par, c, :] = v

            @pl.when(cid == 1)
            def _():
                obuf[par, W, :] = obuf[par, W - 1, :]

            wr(t, par, False)
            wr(t, par, True)
            return 0

        lax.fori_loop(0, total, lvl_body, 0)
        plsc.subcore_barrier()


def _build(tex4):
    kern = pl.kernel(
        _build_body,
        out_type=jax.ShapeDtypeStruct((T_ROWS, 2 * FD), F32),
        mesh=_mesh(),
        compiler_params=_PARAMS,
        scratch_types=[
            pltpu.VMEM((2, 8, 257, FD), F32),
            pltpu.VMEM((2, 2, 258, FD), F32),
            pltpu.VMEM((2, 130, FD), F32),
            pltpu.SemaphoreType.DMA,
            pltpu.SemaphoreType.DMA,
        ],
    )
    return kern(tex4)


BLK = 128
NBLK = NPTS // (NC * NS * BLK)


def _splat(val, dtype=I32):
    return jnp.full((16,), val, dtype)


def _sample_body(pts_ref, table_ref, out_ref,
                 pts_v, idx_v, w_v, rows_v, out_v, sem, gsem):
    cid = lax.axis_index("c")
    sid = lax.axis_index("s")
    wid = sid * NC + cid
    base0 = wid * (NBLK * BLK)

    def phase_a(blk, par):
        b0 = base0 + blk * BLK
        ld = [pltpu.async_copy(pts_ref.at[c, pl.ds(b0, BLK)], pts_v.at[c],
                               sem) for c in range(4)]
        for h in ld:
            h.wait()

        def grp_a(g, _):
            s = g * 16
            xv = pts_v[0, pl.ds(s, 16)]
            yv = pts_v[1, pl.ds(s, 16)]
            zv = pts_v[2, pl.ds(s, 16)]
            lv = pts_v[3, pl.ds(s, 16)]
            lvc = jnp.minimum(jnp.maximum(lv, 0.0), 7.0)
            m0 = lvc.astype(I32)
            fl = lvc - m0.astype(F32)
            m1 = jnp.minimum(m0 + 1, 7)
            for mipslot, (mm, wm) in enumerate(((m0, 1.0 - fl), (m1, fl))):
                S = jnp.right_shift(_splat(PS), mm)
                Sf = S.astype(F32)
                Sm1 = S - 1
                mbv = _splat(0)
                for m in range(1, N_LEVELS):
                    mbv = jnp.where(mm == m, MB[m], mbv)
                cc = []
                for u in (xv, yv, zv):
                    pos = u * Sf - 0.5
                    ti = pos.astype(I32)
                    tf = ti.astype(F32)
                    neg = tf > pos
                    f0 = jnp.where(neg, ti - 1, ti)
                    f0f = jnp.where(neg, tf - 1.0, tf)
                    frac = pos - f0f
                    e0 = jnp.minimum(jnp.maximum(f0, 0), Sm1)
                    e1 = jnp.minimum(e0 + 1, Sm1)
                    cc.append((e0, e1, frac))
                for p, (ui, vi) in enumerate(((1, 2), (0, 2), (0, 1))):
                    ux0, ux1, fu = cc[ui]
                    vy0, vy1, fv = cc[vi]
                    pb = mbv + p * PLANE_TEXELS
                    r0 = pb + vy0 * S
                    r1 = pb + vy1 * S
                    wa = (1.0 - fv) * wm
                    wb = fv * wm
                    gu = 1.0 - fu
                    k = p * 8 + mipslot * 4
                    q = p * 4 + mipslot * 2
                    idx_v[par, q + 0, pl.ds(s, 16)] = r0 + ux0
                    idx_v[par, q + 1, pl.ds(s, 16)] = r1 + ux0
                    w_v[par, k + 0, pl.ds(s, 16)] = gu * wa
                    w_v[par, k + 1, pl.ds(s, 16)] = fu * wa
                    w_v[par, k + 2, pl.ds(s, 16)] = gu * wb
                    w_v[par, k + 3, pl.ds(s, 16)] = fu * wb
            return 0

        lax.fori_loop(0, 8, grp_a, 0)

    def fire(par):
        for k in range(12):
            pltpu.async_copy(table_ref.at[idx_v.at[par, k]],
                             rows_v.at[par, k], gsem)

    def drain(par):
        for k in range(12):
            pltpu.make_async_copy(table_ref.at[idx_v.at[par, k]],
                                  rows_v.at[par, k], gsem).wait()

    def phase_b(blk, par):
        def grp_b(g, _):
            s = g * 16
            for p in range(3):
                wk = [w_v[par, p * 8 + j, pl.ds(s, 16)] for j in range(8)]
                for i in range(16):
                    acc = None
                    for qq in range(4):
                        lo = rows_v[par, p * 4 + qq, s + i, pl.ds(0, FD)]
                        hi = rows_v[par, p * 4 + qq, s + i, pl.ds(FD, FD)]
                        term = wk[2 * qq][i] * lo + wk[2 * qq + 1][i] * hi
                        acc = term if acc is None else acc + term
                    out_v[s + i, pl.ds(p * 16, 16)] = acc
            return 0

        lax.fori_loop(0, 8, grp_b, 0)
        b0 = base0 + blk * BLK
        pltpu.async_copy(out_v, out_ref.at[pl.ds(b0, BLK)], sem).wait()

    phase_a(0, 0)
    fire(0)

    def pair(i, _):
        blk0 = 2 * i
        phase_a(blk0 + 1, 1)
        fire(1)
        drain(0)
        phase_b(blk0, 0)

        @pl.when(i < NBLK // 2 - 1)
        def _():
            phase_a(blk0 + 2, 0)
            fire(0)

        drain(1)
        phase_b(blk0 + 1, 1)
        return 0

    lax.fori_loop(0, NBLK // 2, pair, 0)


def _sample(pts, table):
    kern = pl.kernel(
        _sample_body,
        out_type=jax.ShapeDtypeStruct((NPTS, 48), F32),
        mesh=_mesh(),
        compiler_params=_PARAMS,
        scratch_types=[
            pltpu.VMEM((4, BLK), F32),
            pltpu.VMEM((2, 12, BLK), I32),
            pltpu.VMEM((2, 24, BLK), F32),
            pltpu.VMEM((2, 12, BLK, 2 * FD), F32),
            pltpu.VMEM((BLK, 48), F32),
            pltpu.SemaphoreType.DMA,
            pltpu.SemaphoreType.DMA,
        ],
    )
    return kern(pts, table)


@jax.jit
def kernel(x, level, texture):
    pts = jnp.concatenate([x.T, level[None, :]], axis=0).astype(F32)
    table = _build(texture)
    return _sample(pts, table)


# speedup vs baseline: 1.2511x; 1.0129x over previous
import functools

import jax
import jax.numpy as jnp
from jax import lax
from jax.experimental import pallas as pl
from jax.experimental.pallas import tpu as pltpu
from jax.experimental.pallas import tpu_sc as plsc

N_LEVELS = 8
PS = 512
FD = 16
NPTS = 262144
MB = (0, 262144, 327680, 344064, 348160, 349184, 349440, 349504)
PLANE_TEXELS = 349520
T_ROWS = 3 * PLANE_TEXELS
NC = 2
NS = 16
F32 = jnp.float32
I32 = jnp.int32


def _mesh():
    return plsc.VectorSubcoreMesh(core_axis_name="c", subcore_axis_name="s")


_PARAMS = pltpu.CompilerParams(use_tc_tiling_on_sc=False,
                               needs_layout_passes=False)


def _build_body(tex_ref, table_ref, tbuf, rbuf, obuf, rsem, wsem):
    cid = lax.axis_index("c")
    sid = lax.axis_index("s")


    def l0_read(t, par):
        p = t // 4
        y0 = sid * 32 + (t - p * 4) * 8
        pltpu.async_copy(
            tex_ref.at[p, pl.ds(y0, 8), pl.ds(cid * 256, 256)],
            tbuf.at[par, :, pl.ds(0, 256)], rsem)
        ecol = jnp.minimum(cid * 256 + 256, 511)
        pltpu.async_copy(
            tex_ref.at[p, pl.ds(y0, 8), pl.ds(ecol, 1)],
            tbuf.at[par, :, pl.ds(256, 1)], rsem)

    def l0_rwait(t, par):
        p = t // 4
        y0 = sid * 32 + (t - p * 4) * 8
        pltpu.make_async_copy(
            tex_ref.at[p, pl.ds(y0, 8), pl.ds(cid * 256, 256)],
            tbuf.at[par, :, pl.ds(0, 256)], rsem).wait()
        pltpu.make_async_copy(
            tex_ref.at[p, pl.ds(y0, 8), pl.ds(0, 1)],
            tbuf.at[par, :, pl.ds(256, 1)], rsem).wait()

    def l0_wfire(t, par, drain):
        p = t // 4
        y0 = sid * 32 + (t - p * 4) * 8
        for r in range(8):
            dst = p * PLANE_TEXELS + (y0 + r) * 512 + cid * 256
            for sl, c0 in ((0, 0), (FD, 1)):
                cp = pltpu.make_async_copy(
                    tbuf.at[par, r, pl.ds(c0, 256)],
                    table_ref.at[pl.ds(dst, 256), pl.ds(sl, FD)], wsem)
                if drain:
                    cp.wait()
                else:
                    cp.start()

    l0_read(0, 0)

    def l0_body(t, _):
        par = t & 1

        @pl.when(t > 0)
        def _():
            l0_wfire(t - 1, 1 - par, True)

        @pl.when(t < 11)
        def _():
            l0_read(t + 1, 1 - par)

        l0_rwait(t, par)
        l0_wfire(t, par, False)
        return 0

    lax.fori_loop(0, 12, l0_body, 0)
    l0_wfire(11, 1, True)
    plsc.subcore_barrier()

    for m in range(1, N_LEVELS):
        S = PS >> m
        W = S // 2
        Sin = S * 2
        rpt = max(S // NS, 1)
        lo = jnp.minimum(sid * rpt, S)
        hi = jnp.minimum((sid + 1) * rpt, S)
        n = hi - lo
        total = 3 * n

        def py_of(t, n=n, lo=lo):
            p = t // jnp.maximum(n, 1)
            return p, lo + t - p * n

        def rd(t, par, go, S=S, Sin=Sin, m=m):
            p, y = py_of(t)
            pbin = p * PLANE_TEXELS + MB[m - 1]
            src0 = pbin + (2 * y) * Sin + cid * S
            for half in range(2):
                cp = pltpu.make_async_copy(
                    table_ref.at[pl.ds(src0 + half * Sin, S + 2),
                                 pl.ds(0, FD)],
                    rbuf.at[par, half, pl.ds(0, S + 2)], rsem)
                if go:
                    cp.start()
                else:
                    cp.wait()

        def wr(t, par, drain, S=S, W=W, m=m):
            p, y = py_of(t)
            dst = p * PLANE_TEXELS + MB[m] + y * S + cid * W
            for sl, c0 in ((0, 0), (FD, 1)):
                cp = pltpu.make_async_copy(
                    obuf.at[par, pl.ds(c0, W)],
                    table_ref.at[pl.ds(dst, W), pl.ds(sl, FD)], wsem)
                if drain:
                    cp.wait()
                else:
                    cp.start()

        def lvl_body(t, _, S=S, W=W):
            par = 0

            rd(t, par, True)

            @pl.when(t > 0)
            def _():
                wr(t - 1, par, True)

            rd(t, par, False)
            for c in range(W + 1):
                v = (rbuf[par, 0, 2 * c, :] + rbuf[par, 1, 2 * c, :]
                     + rbuf[par, 0, 2 * c + 1, :]
                     + rbuf[par, 1, 2 * c + 1, :]) * 0.25
                obuf[par, c, :] = v

            @pl.when(cid == 1)
            def _():
                obuf[par, W, :] = obuf[par, W - 1, :]

            wr(t, par, False)
            return 0

        lax.fori_loop(0, total, lvl_body, 0)

        @pl.when(total > 0)
        def _():
            wr(total - 1, 0, True)

        plsc.subcore_barrier()


def _build(tex4):
    kern = pl.kernel(
        _build_body,
        out_type=jax.ShapeDtypeStruct((T_ROWS, 2 * FD), F32),
        mesh=_mesh(),
        compiler_params=_PARAMS,
        scratch_types=[
            pltpu.VMEM((2, 8, 257, FD), F32),
            pltpu.VMEM((2, 2, 258, FD), F32),
            pltpu.VMEM((2, 130, FD), F32),
            pltpu.SemaphoreType.DMA,
            pltpu.SemaphoreType.DMA,
        ],
    )
    return kern(tex4)


BLK = 128
NBLK = NPTS // (NC * NS * BLK)


def _splat(val, dtype=I32):
    return jnp.full((16,), val, dtype)


def _sample_body(pts_ref, table_ref, out_ref,
                 pts_v, idx_v, w_v, rows_v, out_v, sem, gsem):
    cid = lax.axis_index("c")
    sid = lax.axis_index("s")
    wid = sid * NC + cid
    base0 = wid * (NBLK * BLK)

    def phase_a(blk, par):
        b0 = base0 + blk * BLK
        ld = [pltpu.async_copy(pts_ref.at[c, pl.ds(b0, BLK)], pts_v.at[c],
                               sem) for c in range(4)]
        for h in ld:
            h.wait()

        def grp_a(g, _):
            s = g * 16
            xv = pts_v[0, pl.ds(s, 16)]
            yv = pts_v[1, pl.ds(s, 16)]
            zv = pts_v[2, pl.ds(s, 16)]
            lv = pts_v[3, pl.ds(s, 16)]
            lvc = jnp.minimum(jnp.maximum(lv, 0.0), 7.0)
            m0 = lvc.astype(I32)
            fl = lvc - m0.astype(F32)
            m1 = jnp.minimum(m0 + 1, 7)
            for mipslot, (mm, wm) in enumerate(((m0, 1.0 - fl), (m1, fl))):
                S = jnp.right_shift(_splat(PS), mm)
                Sf = S.astype(F32)
                Sm1 = S - 1
                mbv = _splat(0)
                for m in range(1, N_LEVELS):
                    mbv = jnp.where(mm == m, MB[m], mbv)
                cc = []
                for u in (xv, yv, zv):
                    pos = u * Sf - 0.5
                    ti = pos.astype(I32)
                    tf = ti.astype(F32)
                    neg = tf > pos
                    f0 = jnp.where(neg, ti - 1, ti)
                    f0f = jnp.where(neg, tf - 1.0, tf)
                    frac = pos - f0f
                    e0 = jnp.minimum(jnp.maximum(f0, 0), Sm1)
                    e1 = jnp.minimum(e0 + 1, Sm1)
                    cc.append((e0, e1, frac))
                for p, (ui, vi) in enumerate(((1, 2), (0, 2), (0, 1))):
                    ux0, ux1, fu = cc[ui]
                    vy0, vy1, fv = cc[vi]
                    pb = mbv + p * PLANE_TEXELS
                    r0 = pb + vy0 * S
                    r1 = pb + vy1 * S
                    wa = (1.0 - fv) * wm
                    wb = fv * wm
                    gu = 1.0 - fu
                    k = p * 8 + mipslot * 4
                    q = p * 4 + mipslot * 2
                    idx_v[par, q + 0, pl.ds(s, 16)] = r0 + ux0
                    idx_v[par, q + 1, pl.ds(s, 16)] = r1 + ux0
                    w_v[par, k + 0, pl.ds(s, 16)] = gu * wa
                    w_v[par, k + 1, pl.ds(s, 16)] = fu * wa
                    w_v[par, k + 2, pl.ds(s, 16)] = gu * wb
                    w_v[par, k + 3, pl.ds(s, 16)] = fu * wb
            return 0

        lax.fori_loop(0, 8, grp_a, 0)

    def fire(par):
        for k in range(12):
            pltpu.async_copy(table_ref.at[idx_v.at[par, k]],
                             rows_v.at[par, k], gsem)

    def drain(par):
        for k in range(12):
            pltpu.make_async_copy(table_ref.at[idx_v.at[par, k]],
                                  rows_v.at[par, k], gsem).wait()

    def phase_b(blk, par):
        def grp_b(g, _):
            s = g * 16
            for p in range(3):
                wk = [w_v[par, p * 8 + j, pl.ds(s, 16)] for j in range(8)]
                for i in range(16):
                    acc = None
                    for qq in range(4):
                        lo = rows_v[par, p * 4 + qq, s + i, pl.ds(0, FD)]
                        hi = rows_v[par, p * 4 + qq, s + i, pl.ds(FD, FD)]
                        term = wk[2 * qq][i] * lo + wk[2 * qq + 1][i] * hi
                        acc = term if acc is None else acc + term
                    out_v[s + i, pl.ds(p * 16, 16)] = acc
            return 0

        lax.fori_loop(0, 8, grp_b, 0)
        b0 = base0 + blk * BLK
        pltpu.async_copy(out_v, out_ref.at[pl.ds(b0, BLK)], sem).wait()

    phase_a(0, 0)
    fire(0)

    def pair(i, _):
        blk0 = 2 * i
        phase_a(blk0 + 1, 1)
        fire(1)
        drain(0)
        phase_b(blk0, 0)

        @pl.when(i < NBLK // 2 - 1)
        def _():
            phase_a(blk0 + 2, 0)
            fire(0)

        drain(1)
        phase_b(blk0 + 1, 1)
        return 0

    lax.fori_loop(0, NBLK // 2, pair, 0)


def _sample(pts, table):
    kern = pl.kernel(
        _sample_body,
        out_type=jax.ShapeDtypeStruct((NPTS, 48), F32),
        mesh=_mesh(),
        compiler_params=_PARAMS,
        scratch_types=[
            pltpu.VMEM((4, BLK), F32),
            pltpu.VMEM((2, 12, BLK), I32),
            pltpu.VMEM((2, 24, BLK), F32),
            pltpu.VMEM((2, 12, BLK, 2 * FD), F32),
            pltpu.VMEM((BLK, 48), F32),
            pltpu.SemaphoreType.DMA,
            pltpu.SemaphoreType.DMA,
        ],
    )
    return kern(pts, table)


@jax.jit
def kernel(x, level, texture):
    pts = jnp.concatenate([x.T, level[None, :]], axis=0).astype(F32)
    table = _build(texture)
    return _sample(pts, table)


# speedup vs baseline: 1.2513x; 1.0002x over previous
import functools

import jax
import jax.numpy as jnp
from jax import lax
from jax.experimental import pallas as pl
from jax.experimental.pallas import tpu as pltpu
from jax.experimental.pallas import tpu_sc as plsc

N_LEVELS = 8
PS = 512
FD = 16
NPTS = 262144
MB = (0, 262144, 327680, 344064, 348160, 349184, 349440, 349504)
PLANE_TEXELS = 349520
T_ROWS = 3 * PLANE_TEXELS
NC = 2
NS = 16
F32 = jnp.float32
I32 = jnp.int32


def _mesh():
    return plsc.VectorSubcoreMesh(core_axis_name="c", subcore_axis_name="s")


_PARAMS = pltpu.CompilerParams(use_tc_tiling_on_sc=False,
                               needs_layout_passes=False)


def _build_body(tex_ref, table_ref, tbuf, rbuf, obuf, rsem, wsem):
    cid = lax.axis_index("c")
    sid = lax.axis_index("s")


    def l0_read(t, par):
        p = t // 4
        y0 = sid * 32 + (t - p * 4) * 8
        pltpu.async_copy(
            tex_ref.at[p, pl.ds(y0, 8), pl.ds(cid * 256, 256)],
            tbuf.at[par, :, pl.ds(0, 256)], rsem)
        ecol = jnp.minimum(cid * 256 + 256, 511)
        pltpu.async_copy(
            tex_ref.at[p, pl.ds(y0, 8), pl.ds(ecol, 1)],
            tbuf.at[par, :, pl.ds(256, 1)], rsem)

    def l0_rwait(t, par):
        p = t // 4
        y0 = sid * 32 + (t - p * 4) * 8
        pltpu.make_async_copy(
            tex_ref.at[p, pl.ds(y0, 8), pl.ds(cid * 256, 256)],
            tbuf.at[par, :, pl.ds(0, 256)], rsem).wait()
        pltpu.make_async_copy(
            tex_ref.at[p, pl.ds(y0, 8), pl.ds(0, 1)],
            tbuf.at[par, :, pl.ds(256, 1)], rsem).wait()

    def l0_wfire(t, par, drain):
        p = t // 4
        y0 = sid * 32 + (t - p * 4) * 8
        for r in range(8):
            dst = p * PLANE_TEXELS + (y0 + r) * 512 + cid * 256
            for sl, c0 in ((0, 0), (FD, 1)):
                cp = pltpu.make_async_copy(
                    tbuf.at[par, r, pl.ds(c0, 256)],
                    table_ref.at[pl.ds(dst, 256), pl.ds(sl, FD)], wsem)
                if drain:
                    cp.wait()
                else:
                    cp.start()

    l0_read(0, 0)

    def l0_body(t, _):
        par = t & 1

        @pl.when(t > 0)
        def _():
            l0_wfire(t - 1, 1 - par, True)

        @pl.when(t < 11)
        def _():
            l0_read(t + 1, 1 - par)

        l0_rwait(t, par)
        l0_wfire(t, par, False)
        return 0

    lax.fori_loop(0, 12, l0_body, 0)
    l0_wfire(11, 1, True)
    plsc.subcore_barrier()

    for m in range(1, N_LEVELS):
        S = PS >> m
        W = S // 2
        Sin = S * 2
        rpt = max(S // NS, 1)
        lo = jnp.minimum(sid * rpt, S)
        hi = jnp.minimum((sid + 1) * rpt, S)
        n = hi - lo
        total = 3 * n

        def py_of(t, n=n, lo=lo):
            p = t // jnp.maximum(n, 1)
            return p, lo + t - p * n

        def rd(t, par, go, S=S, Sin=Sin, m=m):
            p, y = py_of(t)
            pbin = p * PLANE_TEXELS + MB[m - 1]
            src0 = pbin + (2 * y) * Sin + cid * S
            for half in range(2):
                cp = pltpu.make_async_copy(
                    table_ref.at[pl.ds(src0 + half * Sin, S + 2),
                                 pl.ds(0, FD)],
                    rbuf.at[par, half, pl.ds(0, S + 2)], rsem)
                if go:
                    cp.start()
                else:
                    cp.wait()

        def wr(t, par, drain, S=S, W=W, m=m):
            p, y = py_of(t)
            dst = p * PLANE_TEXELS + MB[m] + y * S + cid * W
            for sl, c0 in ((0, 0), (FD, 1)):
                cp = pltpu.make_async_copy(
                    obuf.at[par, pl.ds(c0, W)],
                    table_ref.at[pl.ds(dst, W), pl.ds(sl, FD)], wsem)
                if drain:
                    cp.wait()
                else:
                    cp.start()

        def lvl_body(t, _, S=S, W=W):
            par = 0

            rd(t, par, True)

            @pl.when(t > 0)
            def _():
                wr(t - 1, par, True)

            rd(t, par, False)
            for c in range(W + 1):
                v = (rbuf[par, 0, 2 * c, :] + rbuf[par, 1, 2 * c, :]
                     + rbuf[par, 0, 2 * c + 1, :]
                     + rbuf[par, 1, 2 * c + 1, :]) * 0.25
                obuf[par, c, :] = v

            @pl.when(cid == 1)
            def _():
                obuf[par, W, :] = obuf[par, W - 1, :]

            wr(t, par, False)
            return 0

        lax.fori_loop(0, total, lvl_body, 0)

        @pl.when(total > 0)
        def _():
            wr(total - 1, 0, True)

        plsc.subcore_barrier()


def _build(tex4):
    kern = pl.kernel(
        _build_body,
        out_type=jax.ShapeDtypeStruct((T_ROWS, 2 * FD), F32),
        mesh=_mesh(),
        compiler_params=_PARAMS,
        scratch_types=[
            pltpu.VMEM((2, 8, 257, FD), F32),
            pltpu.VMEM((2, 2, 258, FD), F32),
            pltpu.VMEM((2, 130, FD), F32),
            pltpu.SemaphoreType.DMA,
            pltpu.SemaphoreType.DMA,
        ],
    )
    return kern(tex4)


BLK = 128
NBLK = NPTS // (NC * NS * BLK)


def _splat(val, dtype=I32):
    return jnp.full((16,), val, dtype)


def _sample_body(pts_ref, table_ref, out_ref,
                 pts_v, idx_v, w_v, rows_v, out_v, sem, gsem, osem):
    cid = lax.axis_index("c")
    sid = lax.axis_index("s")
    wid = sid * NC + cid
    base0 = wid * (NBLK * BLK)

    def phase_a(blk, par):
        b0 = base0 + blk * BLK
        ld = [pltpu.async_copy(pts_ref.at[c, pl.ds(b0, BLK)], pts_v.at[c],
                               sem) for c in range(4)]
        for h in ld:
            h.wait()

        def grp_a(g, _):
            s = g * 16
            xv = pts_v[0, pl.ds(s, 16)]
            yv = pts_v[1, pl.ds(s, 16)]
            zv = pts_v[2, pl.ds(s, 16)]
            lv = pts_v[3, pl.ds(s, 16)]
            lvc = jnp.minimum(jnp.maximum(lv, 0.0), 7.0)
            m0 = lvc.astype(I32)
            fl = lvc - m0.astype(F32)
            m1 = jnp.minimum(m0 + 1, 7)
            for mipslot, (mm, wm) in enumerate(((m0, 1.0 - fl), (m1, fl))):
                S = jnp.right_shift(_splat(PS), mm)
                Sf = S.astype(F32)
                Sm1 = S - 1
                mbv = _splat(0)
                for m in range(1, N_LEVELS):
                    mbv = jnp.where(mm == m, MB[m], mbv)
                cc = []
                for u in (xv, yv, zv):
                    pos = u * Sf - 0.5
                    ti = pos.astype(I32)
                    tf = ti.astype(F32)
                    neg = tf > pos
                    f0 = jnp.where(neg, ti - 1, ti)
                    f0f = jnp.where(neg, tf - 1.0, tf)
                    frac = pos - f0f
                    e0 = jnp.minimum(jnp.maximum(f0, 0), Sm1)
                    e1 = jnp.minimum(e0 + 1, Sm1)
                    cc.append((e0, e1, frac))
                for p, (ui, vi) in enumerate(((1, 2), (0, 2), (0, 1))):
                    ux0, ux1, fu = cc[ui]
                    vy0, vy1, fv = cc[vi]
                    pb = mbv + p * PLANE_TEXELS
                    r0 = pb + vy0 * S
                    r1 = pb + vy1 * S
                    wa = (1.0 - fv) * wm
                    wb = fv * wm
                    gu = 1.0 - fu
                    k = p * 8 + mipslot * 4
                    q = p * 4 + mipslot * 2
                    idx_v[par, q + 0, pl.ds(s, 16)] = r0 + ux0
                    idx_v[par, q + 1, pl.ds(s, 16)] = r1 + ux0
                    w_v[par, k + 0, pl.ds(s, 16)] = gu * wa
                    w_v[par, k + 1, pl.ds(s, 16)] = fu * wa
                    w_v[par, k + 2, pl.ds(s, 16)] = gu * wb
                    w_v[par, k + 3, pl.ds(s, 16)] = fu * wb
            return 0

        lax.fori_loop(0, 8, grp_a, 0)

    def fire(par):
        for k in range(12):
            pltpu.async_copy(table_ref.at[idx_v.at[par, k]],
                             rows_v.at[par, k], gsem)

    def drain(par):
        for k in range(12):
            pltpu.make_async_copy(table_ref.at[idx_v.at[par, k]],
                                  rows_v.at[par, k], gsem).wait()

    def phase_b(blk, par):
        @pl.when(blk > 0)
        def _():
            pb0 = base0 + (blk - 1) * BLK
            pltpu.make_async_copy(out_v.at[1 - par],
                                  out_ref.at[pl.ds(pb0, BLK)], osem).wait()

        def grp_b(g, _):
            s = g * 16
            for p in range(3):
                wk = [w_v[par, p * 8 + j, pl.ds(s, 16)] for j in range(8)]
                for i in range(16):
                    acc = None
                    for qq in range(4):
                        lo = rows_v[par, p * 4 + qq, s + i, pl.ds(0, FD)]
                        hi = rows_v[par, p * 4 + qq, s + i, pl.ds(FD, FD)]
                        term = wk[2 * qq][i] * lo + wk[2 * qq + 1][i] * hi
                        acc = term if acc is None else acc + term
                    out_v[par, s + i, pl.ds(p * 16, 16)] = acc
            return 0

        lax.fori_loop(0, 8, grp_b, 0)
        b0 = base0 + blk * BLK
        pltpu.async_copy(out_v.at[par], out_ref.at[pl.ds(b0, BLK)], osem)

    phase_a(0, 0)
    fire(0)

    def pair(i, _):
        blk0 = 2 * i
        phase_a(blk0 + 1, 1)
        fire(1)
        drain(0)
        phase_b(blk0, 0)

        @pl.when(i < NBLK // 2 - 1)
        def _():
            phase_a(blk0 + 2, 0)
            fire(0)

        drain(1)
        phase_b(blk0 + 1, 1)
        return 0

    lax.fori_loop(0, NBLK // 2, pair, 0)
    pltpu.make_async_copy(
        out_v.at[(NBLK - 1) & 1],
        out_ref.at[pl.ds(base0 + (NBLK - 1) * BLK, BLK)], osem).wait()


def _sample(pts, table):
    kern = pl.kernel(
        _sample_body,
        out_type=jax.ShapeDtypeStruct((NPTS, 48), F32),
        mesh=_mesh(),
        compiler_params=_PARAMS,
        scratch_types=[
            pltpu.VMEM((4, BLK), F32),
            pltpu.VMEM((2, 12, BLK), I32),
            pltpu.VMEM((2, 24, BLK), F32),
            pltpu.VMEM((2, 12, BLK, 2 * FD), F32),
            pltpu.VMEM((2, BLK, 48), F32),
            pltpu.SemaphoreType.DMA,
            pltpu.SemaphoreType.DMA,
            pltpu.SemaphoreType.DMA,
        ],
    )
    return kern(pts, table)


@jax.jit
def kernel(x, level, texture):
    pts = jnp.concatenate([x.T, level[None, :]], axis=0).astype(F32)
    table = _build(texture)
    return _sample(pts, table)
